# baseline (speedup 1.0000x reference)
import functools

import jax
import jax.numpy as jnp
from jax import lax
from jax.experimental import pallas as pl
from jax.experimental.pallas import tpu as pltpu
from jax.experimental.pallas import tpu_sc as plsc

_INFO = plsc.get_sparse_core_info()
_NC = _INFO.num_cores
_NW = _NC * _INFO.num_subcores
_LN = _INFO.num_lanes


def _argmax_body(x_ref, plog_ref, o_ref, plen_ref):
    x = x_ref[...]
    V = x.shape[1]
    mx = jnp.max(x, axis=1, keepdims=True)
    iV = jax.lax.broadcasted_iota(jnp.int32, x.shape, 1)
    o_ref[...] = jnp.min(jnp.where(x == mx, iV, V), axis=1, keepdims=True)

    @pl.when(pl.program_id(0) == 0)
    def _():
        lg = plog_ref[...]
        D = lg.shape[0]
        mxl = jnp.max(lg, axis=0, keepdims=True)
        iD = jax.lax.broadcasted_iota(jnp.int32, lg.shape, 0)
        plen_ref[0] = jnp.min(jnp.where(lg == mxl, iD, D), axis=0, keepdims=True)


def _sqrt_f32(x):
    y = plsc.bitcast((plsc.bitcast(x, jnp.int32) >> 1) + 0x1FBD1DF5, jnp.float32)
    for _ in range(3):
        y = 0.5 * (y + x / y)
    return y


def _make_sc_match(B, L, D, V):
    E = B // _NW
    G = E // _LN
    mesh = plsc.VectorSubcoreMesh(core_axis_name="c", subcore_axis_name="s")

    @functools.partial(
        pl.kernel,
        out_type=[jax.ShapeDtypeStruct((B,), jnp.float32),
                  jax.ShapeDtypeStruct((B,), jnp.float32)],
        mesh=mesh,
        compiler_params=pltpu.CompilerParams(needs_layout_passes=False),
        scratch_types=[
            pltpu.VMEM((L, 128), jnp.int32),
            pltpu.VMEM((L, 128), jnp.int32),
            pltpu.VMEM((128,), jnp.int32),
            pltpu.VMEM((128,), jnp.int32),
            pltpu.VMEM((_LN * V,), jnp.int32),
            pltpu.VMEM((_LN * V,), jnp.int32),
            pltpu.VMEM((128,), jnp.float32),
            pltpu.VMEM((128,), jnp.float32),
            pltpu.SemaphoreType.DMA,
            pltpu.SemaphoreType.DMA,
            pltpu.SemaphoreType.DMA,
            pltpu.SemaphoreType.DMA,
            pltpu.SemaphoreType.DMA,
            pltpu.SemaphoreType.DMA,
        ],
    )
    def sc_match(t_hbm, p_hbm, tl_hbm, plen_hbm, z_hbm, score_hbm, mean_hbm,
                 t_v, p_v, tl_v, plen_sv, hist0_v, hist1_v, sc_v, mn_v,
                 s_t, s_p, s_tl, s_pl, s_z0, s_z1):
        wid = lax.axis_index("s") * _NC + lax.axis_index("c")
        base = wid * E
        tile = base // 128 * 128
        sub = base - tile
        cp_t = pltpu.async_copy(t_hbm.at[:, pl.ds(tile, 128)], t_v, s_t)
        cp_p = pltpu.async_copy(p_hbm.at[:, pl.ds(tile, 128)], p_v, s_p)
        cp_tl = pltpu.async_copy(tl_hbm.at[pl.ds(base, E)],
                                 tl_v.at[pl.ds(0, E)], s_tl)
        cp_pl = pltpu.async_copy(plen_hbm.at[pl.ds(base, E)],
                                 plen_sv.at[pl.ds(0, E)], s_pl)
        cp_z0 = pltpu.async_copy(z_hbm, hist0_v, s_z0)
        cp_z1 = pltpu.async_copy(z_hbm, hist1_v, s_z1)
        lanes = lax.iota(jnp.int32, _LN)
        ones = jnp.ones((_LN,), jnp.int32)
        cp_t.wait()
        cp_tl.wait()
        cp_p.wait()
        cp_pl.wait()

        for g, hist_v in ((0, hist0_v), (1, hist1_v)):
            off = sub + g * _LN
            hbase = lanes * V
            (cp_z0 if g == 0 else cp_z1).wait()
            tl_vec = tl_v[pl.ds(g * _LN, _LN)]
            plen_vec = plen_sv[pl.ds(g * _LN, _LN)]

            for k in range(L):
                tok = t_v[k, pl.ds(off, _LN)]
                m = k < tl_vec
                plsc.addupdate_scatter(hist_v, [hbase + tok], ones, mask=m)

            match_vec = jnp.zeros((_LN,), jnp.int32)
            for k in range(L):
                tok = p_v[k, pl.ds(off, _LN)]
                m = k < plen_vec
                idx = hbase + tok
                cnt = plsc.load_gather(hist_v, [idx])
                hit = m & (cnt > 0)
                plsc.store_scatter(hist_v, [idx], cnt - 1, mask=hit)
                match_vec = match_vec + jnp.where(hit, 1, 0)

            len_t = jnp.clip(tl_vec, 0, L).astype(jnp.float32) + 1e-9
            len_p = jnp.clip(plen_vec, 0, L).astype(jnp.float32) + 1e-9
            prec = match_vec.astype(jnp.float32) / len_p
            ls = jnp.exp(jnp.minimum(0.0, 1.0 - len_t / len_p))
            score = jnp.clip(ls * _sqrt_f32(prec), 0.0, 1.0)
            w = ((tl_vec > 0) & (plen_vec > 0)).astype(jnp.float32)
            sc_v[pl.ds(g * _LN, _LN)] = score
            mn_v[pl.ds(g * _LN, _LN)] = score / w

        pltpu.sync_copy(sc_v.at[pl.ds(0, E)], score_hbm.at[pl.ds(base, E)])
        pltpu.sync_copy(mn_v.at[pl.ds(0, E)], mean_hbm.at[pl.ds(base, E)])

    return sc_match


def kernel(y_true_tokens, y_true_length, y_pred_tokens, y_pred_length):
    B, L, NV, V = y_pred_tokens.shape
    D = y_pred_length.shape[1]
    xT = y_pred_tokens.transpose(1, 2, 3, 0).reshape(L, V, B)
    t2 = y_true_tokens.transpose(1, 2, 0).reshape(L, B)
    plog2 = y_pred_length.transpose(1, 0)
    Lb, Bs = 5, B // 2
    p_tok, plen = pl.pallas_call(
        _argmax_body,
        grid=(L // Lb, B // Bs),
        in_specs=[pl.BlockSpec((Lb, V, Bs), lambda i, j: (i, 0, j)),
                  pl.BlockSpec((D, Bs), lambda i, j: (0, j))],
        out_specs=[pl.BlockSpec((Lb, 1, Bs), lambda i, j: (i, 0, j)),
                   pl.BlockSpec((1, 1, Bs), lambda i, j: (0, 0, j))],
        out_shape=[jax.ShapeDtypeStruct((L, 1, B), jnp.int32),
                   jax.ShapeDtypeStruct((1, 1, B), jnp.int32)],
    )(xT, plog2)
    zeros_hist = jnp.zeros((_LN * V,), jnp.int32)
    sc = _make_sc_match(B, L, D, V)
    score, mean = sc(t2, p_tok.reshape(L, B), y_true_length,
                     plen.reshape(B), zeros_hist)
    return (score.reshape(B, NV), mean.reshape(B, NV))


# speedup vs baseline: 1.0060x; 1.0060x over previous
import functools

import jax
import jax.numpy as jnp
from jax import lax
from jax.experimental import pallas as pl
from jax.experimental.pallas import tpu as pltpu
from jax.experimental.pallas import tpu_sc as plsc

_INFO = plsc.get_sparse_core_info()
_NC = _INFO.num_cores
_NW = _NC * _INFO.num_subcores
_LN = _INFO.num_lanes


def _argmax_body(x_ref, plog_ref, o_ref, plen_ref):
    x = x_ref[...]
    V = x.shape[1]
    mx = jnp.max(x, axis=1, keepdims=True)
    iV = jax.lax.broadcasted_iota(jnp.int32, x.shape, 1)
    o_ref[...] = jnp.min(jnp.where(x == mx, iV, V), axis=1, keepdims=True)

    @pl.when(pl.program_id(0) == 0)
    def _():
        lg = plog_ref[...]
        D = lg.shape[0]
        mxl = jnp.max(lg, axis=0, keepdims=True)
        iD = jax.lax.broadcasted_iota(jnp.int32, lg.shape, 0)
        plen_ref[0] = jnp.min(jnp.where(lg == mxl, iD, D), axis=0, keepdims=True)


def _sqrt_f32(x):
    y = plsc.bitcast((plsc.bitcast(x, jnp.int32) >> 1) + 0x1FBD1DF5, jnp.float32)
    for _ in range(3):
        y = 0.5 * (y + x / y)
    return y


def _make_sc_match(B, L, D, V):
    E = B // _NW
    G = E // _LN
    mesh = plsc.VectorSubcoreMesh(core_axis_name="c", subcore_axis_name="s")

    @functools.partial(
        pl.kernel,
        out_type=[jax.ShapeDtypeStruct((B,), jnp.float32),
                  jax.ShapeDtypeStruct((B,), jnp.float32)],
        mesh=mesh,
        compiler_params=pltpu.CompilerParams(needs_layout_passes=False),
        scratch_types=[
            pltpu.VMEM((L, 128), jnp.int32),
            pltpu.VMEM((L, 128), jnp.int32),
            pltpu.VMEM((128,), jnp.int32),
            pltpu.VMEM((128,), jnp.int32),
            pltpu.VMEM((_LN * V,), jnp.int32),
            pltpu.VMEM((_LN * V,), jnp.int32),
            pltpu.VMEM((128,), jnp.float32),
            pltpu.VMEM((128,), jnp.float32),
            pltpu.SemaphoreType.DMA,
            pltpu.SemaphoreType.DMA,
            pltpu.SemaphoreType.DMA,
            pltpu.SemaphoreType.DMA,
            pltpu.SemaphoreType.DMA,
            pltpu.SemaphoreType.DMA,
        ],
    )
    def sc_match(t_hbm, p_hbm, tl_hbm, plen_hbm, z_hbm, score_hbm, mean_hbm,
                 t_v, p_v, tl_v, plen_sv, hist0_v, hist1_v, sc_v, mn_v,
                 s_t, s_p, s_tl, s_pl, s_z0, s_z1):
        wid = lax.axis_index("s") * _NC + lax.axis_index("c")
        base = wid * E
        tile = base // 128 * 128
        sub = base - tile
        cp_t = pltpu.async_copy(t_hbm.at[:, pl.ds(tile, 128)], t_v, s_t)
        cp_p = pltpu.async_copy(p_hbm.at[:, pl.ds(tile, 128)], p_v, s_p)
        cp_tl = pltpu.async_copy(tl_hbm.at[pl.ds(base, E)],
                                 tl_v.at[pl.ds(0, E)], s_tl)
        cp_pl = pltpu.async_copy(plen_hbm.at[pl.ds(base, E)],
                                 plen_sv.at[pl.ds(0, E)], s_pl)
        cp_z0 = pltpu.async_copy(z_hbm, hist0_v, s_z0)
        cp_z1 = pltpu.async_copy(z_hbm, hist1_v, s_z1)
        lanes = lax.iota(jnp.int32, _LN)
        ones = jnp.ones((_LN,), jnp.int32)
        cp_t.wait()
        cp_tl.wait()
        cp_p.wait()
        cp_pl.wait()

        for g, hist_v in ((0, hist0_v), (1, hist1_v)):
            off = sub + g * _LN
            hbase = lanes * V
            (cp_z0 if g == 0 else cp_z1).wait()
            tl_vec = tl_v[pl.ds(g * _LN, _LN)]
            plen_vec = plen_sv[pl.ds(g * _LN, _LN)]

            for k in range(L):
                tok = t_v[k, pl.ds(off, _LN)]
                m = k < tl_vec
                plsc.addupdate_scatter(hist_v, [hbase + tok], ones, mask=m)

            match_vec = jnp.zeros((_LN,), jnp.int32)
            for k in range(L):
                tok = p_v[k, pl.ds(off, _LN)]
                m = k < plen_vec
                idx = hbase + tok
                cnt = plsc.load_gather(hist_v, [idx])
                hit = m & (cnt > 0)
                plsc.store_scatter(hist_v, [idx], cnt - 1, mask=hit)
                match_vec = match_vec + jnp.where(hit, 1, 0)

            len_t = jnp.clip(tl_vec, 0, L).astype(jnp.float32) + 1e-9
            len_p = jnp.clip(plen_vec, 0, L).astype(jnp.float32) + 1e-9
            prec = match_vec.astype(jnp.float32) / len_p
            ls = jnp.exp(jnp.minimum(0.0, 1.0 - len_t / len_p))
            score = jnp.clip(ls * _sqrt_f32(prec), 0.0, 1.0)
            w = ((tl_vec > 0) & (plen_vec > 0)).astype(jnp.float32)
            sc_v[pl.ds(g * _LN, _LN)] = score
            mn_v[pl.ds(g * _LN, _LN)] = score / w

        pltpu.sync_copy(sc_v.at[pl.ds(0, E)], score_hbm.at[pl.ds(base, E)])
        pltpu.sync_copy(mn_v.at[pl.ds(0, E)], mean_hbm.at[pl.ds(base, E)])

    return sc_match


def kernel(y_true_tokens, y_true_length, y_pred_tokens, y_pred_length):
    B, L, NV, V = y_pred_tokens.shape
    D = y_pred_length.shape[1]
    xT = y_pred_tokens.transpose(1, 2, 3, 0).reshape(L, V, B)
    t2 = y_true_tokens.transpose(1, 2, 0).reshape(L, B)
    plog2 = y_pred_length.transpose(1, 0)
    Lb = 5
    p_tok, plen = pl.pallas_call(
        _argmax_body,
        grid=(L // Lb,),
        in_specs=[pl.BlockSpec((Lb, V, B), lambda i: (i, 0, 0)),
                  pl.BlockSpec((D, B), lambda i: (0, 0))],
        out_specs=[pl.BlockSpec((Lb, 1, B), lambda i: (i, 0, 0)),
                   pl.BlockSpec((1, 1, B), lambda i: (0, 0, 0))],
        out_shape=[jax.ShapeDtypeStruct((L, 1, B), jnp.int32),
                   jax.ShapeDtypeStruct((1, 1, B), jnp.int32)],
    )(xT, plog2)
    zeros_hist = jnp.zeros((_LN * V,), jnp.int32)
    sc = _make_sc_match(B, L, D, V)
    score, mean = sc(t2, p_tok.reshape(L, B), y_true_length,
                     plen.reshape(B), zeros_hist)
    return (score.reshape(B, NV), mean.reshape(B, NV))


# speedup vs baseline: 1.0299x; 1.0238x over previous
import functools

import jax
import jax.numpy as jnp
from jax import lax
from jax.experimental import pallas as pl
from jax.experimental.pallas import tpu as pltpu
from jax.experimental.pallas import tpu_sc as plsc

_INFO = plsc.get_sparse_core_info()
_NC = _INFO.num_cores
_NW = _NC * _INFO.num_subcores
_LN = _INFO.num_lanes


def _argmax_body(xa_ref, xb_ref, plog_ref, oa_ref, ob_ref, plen_ref):
    def _am(x_ref, o_ref):
        x = x_ref[...]
        V = x.shape[1]
        mx = jnp.max(x, axis=1, keepdims=True)
        iV = jax.lax.broadcasted_iota(jnp.int32, x.shape, 1)
        o_ref[...] = jnp.min(jnp.where(x == mx, iV, V), axis=1, keepdims=True)
    _am(xa_ref, oa_ref)
    _am(xb_ref, ob_ref)

    @pl.when(pl.program_id(0) == 0)
    def _():
        lg = plog_ref[...]
        D = lg.shape[0]
        mxl = jnp.max(lg, axis=0, keepdims=True)
        iD = jax.lax.broadcasted_iota(jnp.int32, lg.shape, 0)
        plen_ref[0] = jnp.min(jnp.where(lg == mxl, iD, D), axis=0, keepdims=True)


def _sqrt_f32(x):
    y = plsc.bitcast((plsc.bitcast(x, jnp.int32) >> 1) + 0x1FBD1DF5, jnp.float32)
    for _ in range(3):
        y = 0.5 * (y + x / y)
    return y


def _make_sc_match(B, L, D, V):
    E = B // _NW
    G = E // _LN
    mesh = plsc.VectorSubcoreMesh(core_axis_name="c", subcore_axis_name="s")

    @functools.partial(
        pl.kernel,
        out_type=[jax.ShapeDtypeStruct((B,), jnp.float32),
                  jax.ShapeDtypeStruct((B,), jnp.float32)],
        mesh=mesh,
        compiler_params=pltpu.CompilerParams(needs_layout_passes=False),
        scratch_types=[
            pltpu.VMEM((L, 128), jnp.int32),
            pltpu.VMEM((L, 128), jnp.int32),
            pltpu.VMEM((128,), jnp.int32),
            pltpu.VMEM((128,), jnp.int32),
            pltpu.VMEM((_LN * V,), jnp.int32),
            pltpu.VMEM((_LN * V,), jnp.int32),
            pltpu.VMEM((128,), jnp.float32),
            pltpu.VMEM((128,), jnp.float32),
            pltpu.SemaphoreType.DMA,
            pltpu.SemaphoreType.DMA,
            pltpu.SemaphoreType.DMA,
            pltpu.SemaphoreType.DMA,
            pltpu.SemaphoreType.DMA,
            pltpu.SemaphoreType.DMA,
        ],
    )
    def sc_match(t_hbm, p_hbm, tl_hbm, plen_hbm, z_hbm, score_hbm, mean_hbm,
                 t_v, p_v, tl_v, plen_sv, hist0_v, hist1_v, sc_v, mn_v,
                 s_t, s_p, s_tl, s_pl, s_z0, s_z1):
        wid = lax.axis_index("s") * _NC + lax.axis_index("c")
        base = wid * E
        tile = base // 128 * 128
        sub = base - tile
        cp_t = pltpu.async_copy(t_hbm.at[:, pl.ds(tile, 128)], t_v, s_t)
        cp_p = pltpu.async_copy(p_hbm.at[:, pl.ds(tile, 128)], p_v, s_p)
        cp_tl = pltpu.async_copy(tl_hbm.at[pl.ds(base, E)],
                                 tl_v.at[pl.ds(0, E)], s_tl)
        cp_pl = pltpu.async_copy(plen_hbm.at[pl.ds(base, E)],
                                 plen_sv.at[pl.ds(0, E)], s_pl)
        cp_z0 = pltpu.async_copy(z_hbm, hist0_v, s_z0)
        cp_z1 = pltpu.async_copy(z_hbm, hist1_v, s_z1)
        lanes = lax.iota(jnp.int32, _LN)
        ones = jnp.ones((_LN,), jnp.int32)
        cp_t.wait()
        cp_tl.wait()
        cp_p.wait()
        cp_pl.wait()

        for g, hist_v in ((0, hist0_v), (1, hist1_v)):
            off = sub + g * _LN
            hbase = lanes * V
            (cp_z0 if g == 0 else cp_z1).wait()
            tl_vec = tl_v[pl.ds(g * _LN, _LN)]
            plen_vec = plen_sv[pl.ds(g * _LN, _LN)]

            for k in range(L):
                tok = t_v[k, pl.ds(off, _LN)]
                m = k < tl_vec
                plsc.addupdate_scatter(hist_v, [hbase + tok], ones, mask=m)

            match_vec = jnp.zeros((_LN,), jnp.int32)
            for k in range(L):
                tok = p_v[k, pl.ds(off, _LN)]
                m = k < plen_vec
                idx = hbase + tok
                cnt = plsc.load_gather(hist_v, [idx])
                hit = m & (cnt > 0)
                plsc.store_scatter(hist_v, [idx], cnt - 1, mask=hit)
                match_vec = match_vec + jnp.where(hit, 1, 0)

            len_t = jnp.clip(tl_vec, 0, L).astype(jnp.float32) + 1e-9
            len_p = jnp.clip(plen_vec, 0, L).astype(jnp.float32) + 1e-9
            prec = match_vec.astype(jnp.float32) / len_p
            ls = jnp.exp(jnp.minimum(0.0, 1.0 - len_t / len_p))
            score = jnp.clip(ls * _sqrt_f32(prec), 0.0, 1.0)
            w = ((tl_vec > 0) & (plen_vec > 0)).astype(jnp.float32)
            sc_v[pl.ds(g * _LN, _LN)] = score
            mn_v[pl.ds(g * _LN, _LN)] = score / w

        pltpu.sync_copy(sc_v.at[pl.ds(0, E)], score_hbm.at[pl.ds(base, E)])
        pltpu.sync_copy(mn_v.at[pl.ds(0, E)], mean_hbm.at[pl.ds(base, E)])

    return sc_match


def kernel(y_true_tokens, y_true_length, y_pred_tokens, y_pred_length):
    B, L, NV, V = y_pred_tokens.shape
    D = y_pred_length.shape[1]
    xT = y_pred_tokens.transpose(1, 2, 3, 0).reshape(L, V, B)
    t2 = y_true_tokens.transpose(1, 2, 0).reshape(L, B)
    plog2 = y_pred_length.transpose(1, 0)
    Lb = 2
    half = (L // 2) // Lb
    p_a, p_b, plen = pl.pallas_call(
        _argmax_body,
        grid=(half,),
        in_specs=[pl.BlockSpec((Lb, V, B), lambda i: (i, 0, 0)),
                  pl.BlockSpec((Lb, V, B), lambda i: (i + (L // 2) // Lb, 0, 0)),
                  pl.BlockSpec((D, B), lambda i: (0, 0))],
        out_specs=[pl.BlockSpec((Lb, 1, B), lambda i: (i, 0, 0)),
                   pl.BlockSpec((Lb, 1, B), lambda i: (i, 0, 0)),
                   pl.BlockSpec((1, 1, B), lambda i: (0, 0, 0))],
        out_shape=[jax.ShapeDtypeStruct((L // 2, 1, B), jnp.int32),
                   jax.ShapeDtypeStruct((L // 2, 1, B), jnp.int32),
                   jax.ShapeDtypeStruct((1, 1, B), jnp.int32)],
    )(xT, xT, plog2)
    p_tok = jnp.concatenate([p_a, p_b], axis=0)
    zeros_hist = jnp.zeros((_LN * V,), jnp.int32)
    sc = _make_sc_match(B, L, D, V)
    score, mean = sc(t2, p_tok.reshape(L, B), y_true_length,
                     plen.reshape(B), zeros_hist)
    return (score.reshape(B, NV), mean.reshape(B, NV))
